# Initial kernel scaffold; baseline (speedup 1.0000x reference)
import jax
import jax.numpy as jnp
from jax.experimental import pallas as pl


def kernel(prediction, groundtruth, anchors, seen):
    raise NotImplementedError("write your pallas kernel here")


# speedup vs baseline: 12.9137x; 12.9137x over previous
import jax
import jax.numpy as jnp
from jax.experimental import pallas as pl
from jax.experimental.pallas import tpu as pltpu

NA = 5
NC = 20
NGT = 20
GY = 52
GX = 52
NPOS = GY * GX
CH = 25
IOU_TH = 0.6
L_OBJ = 5.0
L_PRIOR = 0.01
EPS = 1e-05
BIG = 1e9


def _body(pred_ref, gt_ref, gtt_ref, cxy_ref, anch_ref, seen_ref, out_ref,
          bo_ref, bi_ref):
    b = pl.program_id(0)

    @pl.when(b == 0)
    def _():
        out_ref[0, 0] = 0.0

    cx = cxy_ref[0:1, :]
    cy = cxy_ref[1:2, :]

    g = gt_ref[0]
    g0 = g[:, 0:1]
    g1 = g[:, 1:2]
    g2 = g[:, 2:3]
    g3 = g[:, 3:4]
    gx1 = g0 - g2 * 0.5
    gy1 = g1 - g3 * 0.5
    gx2 = g0 + g2 * 0.5
    gy2 = g1 + g3 * 0.5
    area_g = (gx2 - gx1) * (gy2 - gy1)

    j_iota = jax.lax.broadcasted_iota(jnp.int32, (NGT, NPOS), 0).astype(jnp.float32)
    pos_iota = jax.lax.broadcasted_iota(jnp.int32, (NGT, NPOS), 1).astype(jnp.float32)
    pos_row = jax.lax.broadcasted_iota(jnp.int32, (1, NPOS), 1).astype(jnp.float32)

    gbest = jnp.full((NGT, 1), -1.0, jnp.float32)
    gflat = jnp.zeros((NGT, 1), jnp.float32)
    for a in range(NA):
        aw = anch_ref[a, 0]
        ah = anch_ref[a, 1]
        px1 = cx - aw * 0.5
        py1 = cy - ah * 0.5
        px2 = cx + aw * 0.5
        py2 = cy + ah * 0.5
        area_p = (px2 - px1) * (py2 - py1)
        iw = jnp.maximum(jnp.minimum(px2, gx2) - jnp.maximum(px1, gx1), 0.0)
        ih = jnp.maximum(jnp.minimum(py2, gy2) - jnp.maximum(py1, gy1), 0.0)
        inter = iw * ih
        iou = inter / (area_g + area_p - inter + 1e-10)

        bo = jnp.max(iou, axis=0, keepdims=True)
        bi = jnp.min(jnp.where(iou == bo, j_iota, BIG), axis=0,
                     keepdims=True)
        bo_ref[a:a + 1, :] = bo
        bi_ref[a:a + 1, :] = bi

        m = jnp.max(iou, axis=1, keepdims=True)
        pidx = jnp.min(jnp.where(iou == m, pos_iota, BIG), axis=1,
                       keepdims=True)
        flat = pidx + float(a * NPOS)
        upd = m > gbest
        gbest = jnp.where(upd, m, gbest)
        gflat = jnp.where(upd, flat, gflat)

    gT = gtt_ref[0]
    clsrow = gT[4:5, :]
    c_iota = jax.lax.broadcasted_iota(jnp.int32, (NC, NGT), 0).astype(jnp.float32)
    ind = jnp.where(clsrow == c_iota, 1.0, 0.0)
    g_ext = jnp.concatenate([gT, ind], axis=0).astype(jnp.bfloat16)

    seen_lt = (seen_ref[0, 0] < 12800).astype(jnp.float32)
    prior_w = L_PRIOR * seen_lt

    acc = jnp.zeros((1, NPOS), jnp.float32)

    for a in range(NA):
        flat_row = pos_row + float(a * NPOS)
        eqm = gflat == flat_row
        scat_j = jnp.max(jnp.where(eqm, j_iota, -1.0), axis=0,
                         keepdims=True)
        hit = scat_j >= 0.0
        bi_a = jnp.where(hit, scat_j, bi_ref[a:a + 1, :])
        bo_a = jnp.where(hit, 2.0, bo_ref[a:a + 1, :])
        matchf = (bo_a > IOU_TH).astype(jnp.float32)
        negf = 1.0 - matchf

        onehot = jnp.where(j_iota == bi_a, 1.0, 0.0).astype(jnp.bfloat16)
        t = jax.lax.dot_general(g_ext, onehot, (((1,), (0,)), ((), ())),
                                preferred_element_type=jnp.float32)

        base = a * CH
        aw = anch_ref[a, 0]
        ah = anch_ref[a, 1]
        p0 = jax.nn.sigmoid(pred_ref[0, base + 0:base + 1, :])
        p1 = jax.nn.sigmoid(pred_ref[0, base + 1:base + 2, :])
        p2 = jnp.exp(pred_ref[0, base + 2:base + 3, :]) * aw
        p3 = jnp.exp(pred_ref[0, base + 3:base + 4, :]) * ah
        p4 = jax.nn.sigmoid(pred_ref[0, base + 4:base + 5, :])

        t0 = t[0:1, :] * matchf
        t1 = t[1:2, :] * matchf
        t2 = t[2:3, :] * matchf
        t3 = t[3:4, :] * matchf

        iw = jnp.maximum(
            jnp.minimum(p0 + p2 * 0.5, t0 + t2 * 0.5)
            - jnp.maximum(p0 - p2 * 0.5, t0 - t2 * 0.5), 0.0)
        ih = jnp.maximum(
            jnp.minimum(p1 + p3 * 0.5, t1 + t3 * 0.5)
            - jnp.maximum(p1 - p3 * 0.5, t1 - t3 * 0.5), 0.0)
        inter = iw * ih
        iou_pt = inter / (p2 * p3 + t2 * t3 - inter + EPS)

        noobj_c = jnp.where(iou_pt <= IOU_TH, p4 * p4, 0.0)
        prior_c = negf * ((p0 - 0.5 / GX) ** 2 + (p1 - 0.5 / GY) ** 2
                          + (p2 - aw) ** 2 + (p3 - ah) ** 2)
        box_c = matchf * ((p0 - t[0:1, :]) ** 2 + (p1 - t[1:2, :]) ** 2
                          + (p2 - t[2:3, :]) ** 2 + (p3 - t[3:4, :]) ** 2)
        obj_c = matchf * (p4 - iou_pt) ** 2

        clsm = pred_ref[0, base + 5:base + CH, :] * matchf
        mx = jnp.max(clsm, axis=0, keepdims=True)
        e = jnp.exp(clsm - mx)
        sm = e / jnp.sum(e, axis=0, keepdims=True)
        clsdiff = sm - t[5:CH, :] * matchf
        d0 = clsdiff[0:1, :] - negf
        cls_c = (jnp.sum(clsdiff[1:, :] ** 2, axis=0, keepdims=True)
                 + d0 * d0)

        acc = acc + (noobj_c + box_c + L_OBJ * obj_c + cls_c
                     + prior_w * prior_c)

    out_ref[0, 0] += jnp.sum(acc)


def _run(pred_r, gt, gt_t, cxy, anch, seen_arr, interpret=False):
    B = pred_r.shape[0]
    return pl.pallas_call(
        _body,
        grid=(B,),
        in_specs=[
            pl.BlockSpec((1, NA * CH, NPOS), lambda b: (b, 0, 0)),
            pl.BlockSpec((1, NGT, 5), lambda b: (b, 0, 0)),
            pl.BlockSpec((1, 5, NGT), lambda b: (b, 0, 0)),
            pl.BlockSpec((2, NPOS), lambda b: (0, 0)),
            pl.BlockSpec(memory_space=pltpu.SMEM),
            pl.BlockSpec(memory_space=pltpu.SMEM),
        ],
        out_specs=pl.BlockSpec(memory_space=pltpu.SMEM),
        out_shape=jax.ShapeDtypeStruct((1, 1), jnp.float32),
        scratch_shapes=[
            pltpu.VMEM((8, NPOS), jnp.float32),
            pltpu.VMEM((8, NPOS), jnp.float32),
        ],
        interpret=interpret,
    )(pred_r, gt, gt_t, cxy, anch, seen_arr)


def kernel(prediction, groundtruth, anchors, seen, interpret=False):
    B = prediction.shape[0]
    pred_r = prediction.reshape(B, NA * CH, NPOS)
    gt = groundtruth
    gt_t = jnp.transpose(groundtruth, (0, 2, 1))
    xs = (jnp.arange(GX, dtype=jnp.float32) + 0.5) / GX
    ys = (jnp.arange(GY, dtype=jnp.float32) + 0.5) / GY
    cx = jnp.tile(xs, (GY,))
    cy = jnp.repeat(ys, GX)
    cxy = jnp.stack([cx, cy], axis=0)
    anch = anchors.reshape(NA, 2)
    seen_arr = jnp.asarray(seen, jnp.int32).reshape(1, 1)
    out = _run(pred_r, gt, gt_t, cxy, anch, seen_arr, interpret=interpret)
    return out[0, 0]
